# Initial kernel scaffold; baseline (speedup 1.0000x reference)
import jax
import jax.numpy as jnp
from jax.experimental import pallas as pl


def kernel(x, atom_emb, graph_token):
    raise NotImplementedError("write your pallas kernel here")


# speedup vs baseline: 3.4707x; 3.4707x over previous
import jax
import jax.numpy as jnp
from jax import lax
from jax.experimental import pallas as pl
from jax.experimental.pallas import tpu as pltpu
from jax.experimental.pallas import tpu_sc as plsc

N_GRAPH = 128
N_NODE = 256
F = 9
H = 768
NC, NS = 2, 16
NW = NC * NS
GPW = N_GRAPH // NW
K = 8
NB = N_NODE // K
STEPS = GPW * NB
IDX_PER_W = GPW * N_NODE * F
RPB = K * F
OUT_ROWS = N_GRAPH * (N_NODE + 1)


def _sc_body(x_hbm, table_hbm, tok_hbm, out_hbm, idx_v, rows_v, acc_v, tok_v, sem):
    wid = lax.axis_index("s") * NC + lax.axis_index("c")
    pltpu.sync_copy(x_hbm.at[pl.ds(wid * IDX_PER_W, IDX_PER_W)], idx_v)
    pltpu.sync_copy(tok_hbm, tok_v)
    for g in range(GPW):
        pltpu.sync_copy(tok_v, out_hbm.at[pl.ds((wid * GPW + g) * (N_NODE + 1), 1)])

    def start_gather(s, buf):
        pltpu.async_copy(
            table_hbm.at[idx_v.at[pl.ds(s * RPB, RPB)]],
            rows_v.at[pl.ds(buf * RPB, RPB)],
            sem,
        )

    start_gather(0, 0)

    def step(s, carry):
        buf = lax.rem(s, 2)
        pltpu.make_async_copy(
            table_hbm.at[idx_v.at[pl.ds(0, RPB)]],
            rows_v.at[pl.ds(0, RPB)],
            sem,
        ).wait()

        @pl.when(s + 1 < STEPS)
        def _():
            start_gather(s + 1, 1 - buf)

        boff = buf * RPB

        def reduce_cols(j, c):
            col = pl.ds(j * 16, 16)
            for i in range(K):
                a = rows_v[boff + i * F, col]
                for r in range(1, F):
                    a = a + rows_v[boff + i * F + r, col]
                acc_v[i, col] = a
            return c

        lax.fori_loop(0, H // 16, reduce_cols, 0)

        g = lax.div(s, NB)
        b = lax.rem(s, NB)
        row0 = (wid * GPW + g) * (N_NODE + 1) + 1 + b * K
        pltpu.sync_copy(acc_v, out_hbm.at[pl.ds(row0, K)])
        return carry

    lax.fori_loop(0, STEPS, step, 0)


def kernel(x, atom_emb, graph_token):
    x_flat = x.reshape(-1).astype(jnp.int32)
    mesh = plsc.VectorSubcoreMesh(core_axis_name="c", subcore_axis_name="s")
    out = pl.kernel(
        _sc_body,
        out_type=jax.ShapeDtypeStruct((OUT_ROWS, H), jnp.float32),
        mesh=mesh,
        compiler_params=pltpu.CompilerParams(use_tc_tiling_on_sc=False),
        scratch_types=[
            pltpu.VMEM((IDX_PER_W,), jnp.int32),
            pltpu.VMEM((2 * RPB, H), jnp.float32),
            pltpu.VMEM((K, H), jnp.float32),
            pltpu.VMEM((1, H), jnp.float32),
            pltpu.SemaphoreType.DMA,
        ],
    )(x_flat, atom_emb, graph_token)
    return out.reshape(N_GRAPH, N_NODE + 1, H)
